# baseline (speedup 1.0000x reference)
import jax
import jax.numpy as jnp
from jax.experimental import pallas as pl
from jax.experimental.pallas import tpu as pltpu

_MAX_CTX = 32768


def _copy_kernel(src_ref, dst_ref, sem):
    copy = pltpu.make_async_copy(src_ref, dst_ref, sem)
    copy.start()
    copy.wait()


def kernel(h, mem):
    b, l, d = h.shape
    assert b * l == _MAX_CTX
    flat = h.reshape(b * l, d)
    new_mem = pl.pallas_call(
        _copy_kernel,
        out_shape=jax.ShapeDtypeStruct((b * l, d), h.dtype),
        in_specs=[pl.BlockSpec(memory_space=pl.ANY)],
        out_specs=pl.BlockSpec(memory_space=pl.ANY),
        scratch_shapes=[pltpu.SemaphoreType.DMA],
    )(flat)
    return (h, new_mem)


# speedup vs baseline: 1.0012x; 1.0012x over previous
import jax
import jax.numpy as jnp
from jax.experimental import pallas as pl
from jax.experimental.pallas import tpu as pltpu

_MAX_CTX = 32768
_NCHUNK = 16


def _copy_kernel(src_ref, dst_ref, sems):
    rows = src_ref.shape[0]
    chunk = rows // _NCHUNK
    for i in range(_NCHUNK):
        pltpu.make_async_copy(
            src_ref.at[pl.ds(i * chunk, chunk), :],
            dst_ref.at[pl.ds(i * chunk, chunk), :],
            sems.at[i],
        ).start()
    for i in range(_NCHUNK):
        pltpu.make_async_copy(
            src_ref.at[pl.ds(i * chunk, chunk), :],
            dst_ref.at[pl.ds(i * chunk, chunk), :],
            sems.at[i],
        ).wait()


def kernel(h, mem):
    b, l, d = h.shape
    assert b * l == _MAX_CTX
    flat = h.reshape(b * l, d)
    new_mem = pl.pallas_call(
        _copy_kernel,
        out_shape=jax.ShapeDtypeStruct((b * l, d), h.dtype),
        in_specs=[pl.BlockSpec(memory_space=pl.ANY)],
        out_specs=pl.BlockSpec(memory_space=pl.ANY),
        scratch_shapes=[pltpu.SemaphoreType.DMA((_NCHUNK,))],
    )(flat)
    return (h, new_mem)


# speedup vs baseline: 24.6903x; 24.6609x over previous
import jax
import jax.numpy as jnp
from jax.experimental import pallas as pl
from jax.experimental.pallas import tpu as pltpu

_MAX_CTX = 32768
_BLOCK_ROWS = 1024


def _copy_kernel(src_ref, dst_ref):
    dst_ref[...] = src_ref[...]


def kernel(h, mem):
    b, l, d = h.shape
    assert b * l == _MAX_CTX
    flat = h.reshape(b * l, d)
    grid = (b * l) // _BLOCK_ROWS
    new_mem = pl.pallas_call(
        _copy_kernel,
        grid=(grid,),
        in_specs=[pl.BlockSpec((_BLOCK_ROWS, d), lambda i: (i, 0))],
        out_specs=pl.BlockSpec((_BLOCK_ROWS, d), lambda i: (i, 0)),
        out_shape=jax.ShapeDtypeStruct((b * l, d), h.dtype),
    )(flat)
    return (h, new_mem)


# speedup vs baseline: 24.6952x; 1.0002x over previous
import jax
import jax.numpy as jnp
from jax.experimental import pallas as pl
from jax.experimental.pallas import tpu as pltpu

_MAX_CTX = 32768
_BLOCK_ROWS = 1024


def _copy_kernel(src_ref, dst_ref):
    dst_ref[...] = src_ref[...]


def kernel(h, mem):
    b, l, d = h.shape
    assert b * l == _MAX_CTX
    flat = h.reshape(b * l, d)
    grid = (b * l) // _BLOCK_ROWS
    new_mem = pl.pallas_call(
        _copy_kernel,
        grid=(grid,),
        in_specs=[pl.BlockSpec((_BLOCK_ROWS, d), lambda i: (i, 0))],
        out_specs=pl.BlockSpec((_BLOCK_ROWS, d), lambda i: (i, 0)),
        out_shape=jax.ShapeDtypeStruct((b * l, d), h.dtype),
        compiler_params=pltpu.CompilerParams(
            dimension_semantics=("parallel",),
        ),
    )(flat)
    return (h, new_mem)


# speedup vs baseline: 24.7489x; 1.0022x over previous
import jax
import jax.numpy as jnp
from jax.experimental import pallas as pl
from jax.experimental.pallas import tpu as pltpu

_MAX_CTX = 32768
_D = 1024
_NBUF = 16
_CHUNK_ROWS = 512
_NCHUNKS = _MAX_CTX // _CHUNK_ROWS


def _copy_kernel(src_ref, dst_ref, buf, rsem, wsem):
    ngroups = _NCHUNKS // _NBUF
    for g in range(ngroups):
        for b in range(_NBUF):
            c = g * _NBUF + b
            if g > 0:
                pltpu.make_async_copy(
                    buf.at[b], dst_ref.at[pl.ds((c - _NBUF) * _CHUNK_ROWS, _CHUNK_ROWS), :], wsem.at[b]
                ).wait()
            pltpu.make_async_copy(
                src_ref.at[pl.ds(c * _CHUNK_ROWS, _CHUNK_ROWS), :], buf.at[b], rsem.at[b]
            ).start()
        for b in range(_NBUF):
            c = g * _NBUF + b
            pltpu.make_async_copy(
                src_ref.at[pl.ds(c * _CHUNK_ROWS, _CHUNK_ROWS), :], buf.at[b], rsem.at[b]
            ).wait()
            pltpu.make_async_copy(
                buf.at[b], dst_ref.at[pl.ds(c * _CHUNK_ROWS, _CHUNK_ROWS), :], wsem.at[b]
            ).start()
    g = ngroups - 1
    for b in range(_NBUF):
        c = g * _NBUF + b
        pltpu.make_async_copy(
            buf.at[b], dst_ref.at[pl.ds(c * _CHUNK_ROWS, _CHUNK_ROWS), :], wsem.at[b]
        ).wait()


def kernel(h, mem):
    b, l, d = h.shape
    assert b * l == _MAX_CTX and d == _D
    flat = h.reshape(b * l, d)
    new_mem = pl.pallas_call(
        _copy_kernel,
        in_specs=[pl.BlockSpec(memory_space=pl.ANY)],
        out_specs=pl.BlockSpec(memory_space=pl.ANY),
        out_shape=jax.ShapeDtypeStruct((b * l, d), h.dtype),
        scratch_shapes=[
            pltpu.VMEM((_NBUF, _CHUNK_ROWS, _D), h.dtype),
            pltpu.SemaphoreType.DMA((_NBUF,)),
            pltpu.SemaphoreType.DMA((_NBUF,)),
        ],
    )(flat)
    return (h, new_mem)
